# Initial kernel scaffold; baseline (speedup 1.0000x reference)
import jax
import jax.numpy as jnp
from jax.experimental import pallas as pl


def kernel(inputs, codebook):
    raise NotImplementedError("write your pallas kernel here")


# speedup vs baseline: 3.8103x; 3.8103x over previous
import functools

import jax
import jax.numpy as jnp
from jax import lax
from jax.experimental import pallas as pl
from jax.experimental.pallas import tpu as pltpu
from jax.experimental.pallas import tpu_sc as plsc

_N_TOKENS = 4096
_N_CODES = 1024
_DIM = 64
_PAD_DIM = 128

_NC = 2
_NS = 16
_NW = _NC * _NS
_ROWS_PER_W = _N_TOKENS // _NW


_BLK = 256
_N_BLK = _N_TOKENS // _BLK


def _argmin_body(x_ref, cb_ref, idx_ref, cbp_ref):
    x = x_ref[...]
    cb = cb_ref[...]
    dots = lax.dot_general(x, cb, (((1,), (1,)), ((), ())),
                           precision=lax.Precision.HIGHEST,
                           preferred_element_type=jnp.float32)
    cbn_row = lax.dot_general(jnp.ones((1, _DIM), jnp.float32), cb * cb,
                              (((1,), (1,)), ((), ())),
                              precision=lax.Precision.HIGHEST,
                              preferred_element_type=jnp.float32)
    scores = cbn_row - 2.0 * dots
    iota = lax.broadcasted_iota(jnp.int32, scores.shape, 1)
    big_i = jnp.int32(2**30)

    def first_min(s):
        m = jnp.min(s, axis=1, keepdims=True)
        a = jnp.min(jnp.where(s == m, iota, big_i), axis=1, keepdims=True)
        return a

    def exact_dist(a):
        oh = (iota == a).astype(jnp.float32)
        c = lax.dot_general(oh, cb, (((1,), (0,)), ((), ())),
                            precision=lax.Precision.HIGHEST,
                            preferred_element_type=jnp.float32)
        diff = x - c
        return jnp.sqrt(jnp.sum(diff * diff, axis=1, keepdims=True))

    a1 = first_min(scores)
    a2 = first_min(jnp.where(iota == a1, jnp.inf, scores))
    d1 = exact_dist(a1)
    d2 = exact_dist(a2)
    pick2 = (d2 < d1) | ((d2 == d1) & (a2 < a1))
    win = jnp.where(pick2, a2, a1)
    idx_ref[...] = win.reshape(_BLK)

    @pl.when(pl.program_id(0) == 0)
    def _():
        cbp_ref[...] = jnp.concatenate(
            [cb, jnp.zeros((_N_CODES, _PAD_DIM - _DIM), jnp.float32)], axis=1)


_tc_argmin = pl.pallas_call(
    _argmin_body,
    grid=(_N_BLK,),
    in_specs=[
        pl.BlockSpec((_BLK, _DIM), lambda i: (i, 0)),
        pl.BlockSpec((_N_CODES, _DIM), lambda i: (0, 0)),
    ],
    out_specs=(
        pl.BlockSpec((_BLK,), lambda i: (i,)),
        pl.BlockSpec((_N_CODES, _PAD_DIM), lambda i: (0, 0)),
    ),
    out_shape=(
        jax.ShapeDtypeStruct((_N_TOKENS,), jnp.int32),
        jax.ShapeDtypeStruct((_N_CODES, _PAD_DIM), jnp.float32),
    ),
)


@functools.cache
def _sc_gather_fn():
    @functools.partial(
        pl.kernel,
        mesh=plsc.VectorSubcoreMesh(core_axis_name="c", subcore_axis_name="s"),
        out_type=jax.ShapeDtypeStruct((_N_TOKENS, _PAD_DIM), jnp.float32),
        scratch_types=[
            pltpu.VMEM((_ROWS_PER_W,), jnp.int32),
            pltpu.VMEM((_ROWS_PER_W, _PAD_DIM), jnp.float32),
            pltpu.SemaphoreType.DMA,
        ],
    )
    def _sc_gather(cbp_hbm, idx_hbm, out_hbm, idx_v, rows_v, sem):
        wid = lax.axis_index("s") * _NC + lax.axis_index("c")
        base = wid * _ROWS_PER_W
        pltpu.sync_copy(idx_hbm.at[pl.ds(base, _ROWS_PER_W)], idx_v)
        pltpu.async_copy(cbp_hbm.at[idx_v], rows_v, sem).wait()
        pltpu.sync_copy(rows_v, out_hbm.at[pl.ds(base, _ROWS_PER_W)])

    return _sc_gather


def kernel(inputs, codebook):
    idx, cbp = _tc_argmin(inputs, codebook)
    return _sc_gather_fn()(cbp, idx)[:, :_DIM]


# speedup vs baseline: 4.8303x; 1.2677x over previous
import functools

import jax
import jax.numpy as jnp
from jax import lax
from jax.experimental import pallas as pl
from jax.experimental.pallas import tpu as pltpu
from jax.experimental.pallas import tpu_sc as plsc

_N_TOKENS = 4096
_N_CODES = 1024
_DIM = 64
_PAD_DIM = 128

_NC = 2
_NS = 16
_NW = _NC * _NS
_ROWS_PER_W = _N_TOKENS // _NW

_BLK = 256
_N_BLK = _N_TOKENS // _BLK

_HIGHEST = lax.Precision.HIGHEST


def _rank_body(x_ref, cb_ref, a1_ref, a2_ref, cbp_ref, cbn_scr):
    cb = cb_ref[...]

    @pl.when(pl.program_id(0) == 0)
    def _():
        cbn_scr[...] = lax.dot_general(
            jnp.ones((1, _DIM), jnp.float32), cb * cb, (((1,), (1,)), ((), ())),
            precision=_HIGHEST, preferred_element_type=jnp.float32)
        idcol = lax.broadcasted_iota(
            jnp.int32, (_N_CODES, 1), 0).astype(jnp.float32)
        cbp_ref[...] = jnp.concatenate(
            [cb, idcol,
             jnp.zeros((_N_CODES, _PAD_DIM - _DIM - 1), jnp.float32)], axis=1)

    dots = lax.dot_general(x_ref[...], cb, (((1,), (1,)), ((), ())),
                           precision=_HIGHEST,
                           preferred_element_type=jnp.float32)
    scores = cbn_scr[...] - 2.0 * dots
    iota = lax.broadcasted_iota(jnp.int32, scores.shape, 1)
    big_i = jnp.int32(2**30)

    def first_min(s):
        m = jnp.min(s, axis=1, keepdims=True)
        return jnp.min(jnp.where(s == m, iota, big_i), axis=1, keepdims=True)

    a1 = first_min(scores)
    a2 = first_min(jnp.where(iota == a1, jnp.inf, scores))
    a1_ref[...] = a1.reshape(_BLK)
    a2_ref[...] = a2.reshape(_BLK)


_tc_rank = pl.pallas_call(
    _rank_body,
    grid=(_N_BLK,),
    in_specs=[
        pl.BlockSpec((_BLK, _DIM), lambda i: (i, 0)),
        pl.BlockSpec((_N_CODES, _DIM), lambda i: (0, 0)),
    ],
    out_specs=(
        pl.BlockSpec((_BLK,), lambda i: (i,)),
        pl.BlockSpec((_BLK,), lambda i: (i,)),
        pl.BlockSpec((_N_CODES, _PAD_DIM), lambda i: (0, 0)),
    ),
    out_shape=(
        jax.ShapeDtypeStruct((_N_TOKENS,), jnp.int32),
        jax.ShapeDtypeStruct((_N_TOKENS,), jnp.int32),
        jax.ShapeDtypeStruct((_N_CODES, _PAD_DIM), jnp.float32),
    ),
    scratch_shapes=[pltpu.VMEM((1, _N_CODES), jnp.float32)],
)


@functools.cache
def _sc_gather_fn():
    @functools.partial(
        pl.kernel,
        mesh=plsc.VectorSubcoreMesh(core_axis_name="c", subcore_axis_name="s"),
        out_type=(
            jax.ShapeDtypeStruct((_N_TOKENS, _PAD_DIM), jnp.float32),
            jax.ShapeDtypeStruct((_N_TOKENS, _PAD_DIM), jnp.float32),
        ),
        scratch_types=[
            pltpu.VMEM((_ROWS_PER_W,), jnp.int32),
            pltpu.VMEM((_ROWS_PER_W,), jnp.int32),
            pltpu.VMEM((_ROWS_PER_W, _PAD_DIM), jnp.float32),
            pltpu.VMEM((_ROWS_PER_W, _PAD_DIM), jnp.float32),
            pltpu.SemaphoreType.DMA,
            pltpu.SemaphoreType.DMA,
        ],
    )
    def _sc_gather2(cbp_hbm, a1_hbm, a2_hbm, o1_hbm, o2_hbm,
                    i1_v, i2_v, r1_v, r2_v, sem1, sem2):
        wid = lax.axis_index("s") * _NC + lax.axis_index("c")
        base = wid * _ROWS_PER_W
        pltpu.sync_copy(a1_hbm.at[pl.ds(base, _ROWS_PER_W)], i1_v)
        pltpu.sync_copy(a2_hbm.at[pl.ds(base, _ROWS_PER_W)], i2_v)
        cp1 = pltpu.async_copy(cbp_hbm.at[i1_v], r1_v, sem1)
        cp2 = pltpu.async_copy(cbp_hbm.at[i2_v], r2_v, sem2)
        cp1.wait()
        cp2.wait()
        pltpu.sync_copy(r1_v, o1_hbm.at[pl.ds(base, _ROWS_PER_W)])
        pltpu.sync_copy(r2_v, o2_hbm.at[pl.ds(base, _ROWS_PER_W)])

    return _sc_gather2


def _pick_body(x_ref, g1_ref, g2_ref, q_ref):
    x = x_ref[...]
    g1 = g1_ref[...]
    g2 = g2_ref[...]
    c1 = g1[:, :_DIM]
    c2 = g2[:, :_DIM]
    a1 = g1[:, _DIM:_DIM + 1]
    a2 = g2[:, _DIM:_DIM + 1]
    df1 = x - c1
    df2 = x - c2
    d1 = jnp.sqrt(jnp.sum(df1 * df1, axis=1, keepdims=True))
    d2 = jnp.sqrt(jnp.sum(df2 * df2, axis=1, keepdims=True))
    pick2 = (d2 < d1) | ((d2 == d1) & (a2 < a1))
    q_ref[...] = jnp.where(pick2, c2, c1)


_tc_pick = pl.pallas_call(
    _pick_body,
    grid=(_N_BLK,),
    in_specs=[
        pl.BlockSpec((_BLK, _DIM), lambda i: (i, 0)),
        pl.BlockSpec((_BLK, _PAD_DIM), lambda i: (i, 0)),
        pl.BlockSpec((_BLK, _PAD_DIM), lambda i: (i, 0)),
    ],
    out_specs=pl.BlockSpec((_BLK, _DIM), lambda i: (i, 0)),
    out_shape=jax.ShapeDtypeStruct((_N_TOKENS, _DIM), jnp.float32),
)


def kernel(inputs, codebook):
    a1, a2, cbp = _tc_rank(inputs, codebook)
    g1, g2 = _sc_gather_fn()(cbp, a1, a2)
    return _tc_pick(inputs, g1, g2)


# speedup vs baseline: 5.2908x; 1.0953x over previous
import functools

import jax
import jax.numpy as jnp
from jax import lax
from jax.experimental import pallas as pl
from jax.experimental.pallas import tpu as pltpu
from jax.experimental.pallas import tpu_sc as plsc

_N_TOKENS = 4096
_N_CODES = 1024
_DIM = 64
_PAD_DIM = 128

_NC = 2
_NS = 16
_NW = _NC * _NS
_ROWS_PER_W = _N_TOKENS // _NW

_BLK = 1024
_N_BLK = _N_TOKENS // _BLK

_HIGHEST = lax.Precision.HIGHEST


def _rank_body(x_ref, cb_ref, a1_ref, a2_ref, cbp_ref, cbn_scr):
    cb = cb_ref[...]

    @pl.when(pl.program_id(0) == 0)
    def _():
        cbn_scr[...] = lax.dot_general(
            jnp.ones((1, _DIM), jnp.float32), cb * cb, (((1,), (1,)), ((), ())),
            precision=_HIGHEST, preferred_element_type=jnp.float32)
        idcol = lax.broadcasted_iota(
            jnp.int32, (_N_CODES, 1), 0).astype(jnp.float32)
        cbp_ref[...] = jnp.concatenate(
            [cb, idcol,
             jnp.zeros((_N_CODES, _PAD_DIM - _DIM - 1), jnp.float32)], axis=1)

    dots = lax.dot_general(x_ref[...], cb, (((1,), (1,)), ((), ())),
                           precision=_HIGHEST,
                           preferred_element_type=jnp.float32)
    scores = cbn_scr[...] - 2.0 * dots
    iota = lax.broadcasted_iota(jnp.int32, scores.shape, 1)
    big_i = jnp.int32(2**30)

    def first_min(s):
        m = jnp.min(s, axis=1, keepdims=True)
        return jnp.min(jnp.where(s == m, iota, big_i), axis=1, keepdims=True)

    a1 = first_min(scores)
    a2 = first_min(jnp.where(iota == a1, jnp.inf, scores))
    a1_ref[...] = a1.reshape(_BLK)
    a2_ref[...] = a2.reshape(_BLK)


_tc_rank = pl.pallas_call(
    _rank_body,
    grid=(_N_BLK,),
    in_specs=[
        pl.BlockSpec((_BLK, _DIM), lambda i: (i, 0)),
        pl.BlockSpec((_N_CODES, _DIM), lambda i: (0, 0)),
    ],
    out_specs=(
        pl.BlockSpec((_BLK,), lambda i: (i,)),
        pl.BlockSpec((_BLK,), lambda i: (i,)),
        pl.BlockSpec((_N_CODES, _PAD_DIM), lambda i: (0, 0)),
    ),
    out_shape=(
        jax.ShapeDtypeStruct((_N_TOKENS,), jnp.int32),
        jax.ShapeDtypeStruct((_N_TOKENS,), jnp.int32),
        jax.ShapeDtypeStruct((_N_CODES, _PAD_DIM), jnp.float32),
    ),
    scratch_shapes=[pltpu.VMEM((1, _N_CODES), jnp.float32)],
)


@functools.cache
def _sc_gather_fn():
    @functools.partial(
        pl.kernel,
        mesh=plsc.VectorSubcoreMesh(core_axis_name="c", subcore_axis_name="s"),
        out_type=(
            jax.ShapeDtypeStruct((_N_TOKENS, _PAD_DIM), jnp.float32),
            jax.ShapeDtypeStruct((_N_TOKENS, _PAD_DIM), jnp.float32),
        ),
        scratch_types=[
            pltpu.VMEM((_ROWS_PER_W,), jnp.int32),
            pltpu.VMEM((_ROWS_PER_W,), jnp.int32),
            pltpu.VMEM((_ROWS_PER_W, _PAD_DIM), jnp.float32),
            pltpu.VMEM((_ROWS_PER_W, _PAD_DIM), jnp.float32),
            pltpu.SemaphoreType.DMA,
            pltpu.SemaphoreType.DMA,
        ],
    )
    def _sc_gather2(cbp_hbm, a1_hbm, a2_hbm, o1_hbm, o2_hbm,
                    i1_v, i2_v, r1_v, r2_v, sem1, sem2):
        wid = lax.axis_index("s") * _NC + lax.axis_index("c")
        base = wid * _ROWS_PER_W
        pltpu.sync_copy(a1_hbm.at[pl.ds(base, _ROWS_PER_W)], i1_v)
        pltpu.sync_copy(a2_hbm.at[pl.ds(base, _ROWS_PER_W)], i2_v)
        cp1 = pltpu.async_copy(cbp_hbm.at[i1_v], r1_v, sem1)
        cp2 = pltpu.async_copy(cbp_hbm.at[i2_v], r2_v, sem2)
        cp1.wait()
        cp2.wait()
        pltpu.sync_copy(r1_v, o1_hbm.at[pl.ds(base, _ROWS_PER_W)])
        pltpu.sync_copy(r2_v, o2_hbm.at[pl.ds(base, _ROWS_PER_W)])

    return _sc_gather2


def _pick_body(x_ref, g1_ref, g2_ref, q_ref):
    x = x_ref[...]
    g1 = g1_ref[...]
    g2 = g2_ref[...]
    c1 = g1[:, :_DIM]
    c2 = g2[:, :_DIM]
    a1 = g1[:, _DIM:_DIM + 1]
    a2 = g2[:, _DIM:_DIM + 1]
    df1 = x - c1
    df2 = x - c2
    d1 = jnp.sqrt(jnp.sum(df1 * df1, axis=1, keepdims=True))
    d2 = jnp.sqrt(jnp.sum(df2 * df2, axis=1, keepdims=True))
    pick2 = (d2 < d1) | ((d2 == d1) & (a2 < a1))
    q_ref[...] = jnp.where(pick2, c2, c1)


_tc_pick = pl.pallas_call(
    _pick_body,
    grid=(_N_BLK,),
    in_specs=[
        pl.BlockSpec((_BLK, _DIM), lambda i: (i, 0)),
        pl.BlockSpec((_BLK, _PAD_DIM), lambda i: (i, 0)),
        pl.BlockSpec((_BLK, _PAD_DIM), lambda i: (i, 0)),
    ],
    out_specs=pl.BlockSpec((_BLK, _DIM), lambda i: (i, 0)),
    out_shape=jax.ShapeDtypeStruct((_N_TOKENS, _DIM), jnp.float32),
)


def kernel(inputs, codebook):
    a1, a2, cbp = _tc_rank(inputs, codebook)
    g1, g2 = _sc_gather_fn()(cbp, a1, a2)
    return _tc_pick(inputs, g1, g2)


# speedup vs baseline: 5.3333x; 1.0080x over previous
import functools

import jax
import jax.numpy as jnp
from jax import lax
from jax.experimental import pallas as pl
from jax.experimental.pallas import tpu as pltpu
from jax.experimental.pallas import tpu_sc as plsc

_N_TOKENS = 4096
_N_CODES = 1024
_DIM = 64
_PAD_DIM = 128

_NC = 2
_NS = 16
_NW = _NC * _NS
_ROWS_PER_W = _N_TOKENS // _NW

_BLK = 2048
_N_BLK = _N_TOKENS // _BLK

_HIGHEST = lax.Precision.HIGHEST


def _rank_body(x_ref, cb_ref, a1_ref, a2_ref, cbp_ref, cbn_scr):
    cb = cb_ref[...]

    @pl.when(pl.program_id(0) == 0)
    def _():
        cbn_scr[...] = lax.dot_general(
            jnp.ones((1, _DIM), jnp.float32), cb * cb, (((1,), (1,)), ((), ())),
            precision=_HIGHEST, preferred_element_type=jnp.float32)
        idcol = lax.broadcasted_iota(
            jnp.int32, (_N_CODES, 1), 0).astype(jnp.float32)
        cbp_ref[...] = jnp.concatenate(
            [cb, idcol,
             jnp.zeros((_N_CODES, _PAD_DIM - _DIM - 1), jnp.float32)], axis=1)

    dots = lax.dot_general(x_ref[...], cb, (((1,), (1,)), ((), ())),
                           precision=_HIGHEST,
                           preferred_element_type=jnp.float32)
    scores = cbn_scr[...] - 2.0 * dots
    iota = lax.broadcasted_iota(jnp.int32, scores.shape, 1)
    big_i = jnp.int32(2**30)

    def first_min(s):
        m = jnp.min(s, axis=1, keepdims=True)
        return jnp.min(jnp.where(s == m, iota, big_i), axis=1, keepdims=True)

    a1 = first_min(scores)
    a2 = first_min(jnp.where(iota == a1, jnp.inf, scores))
    a1_ref[...] = a1.reshape(_BLK)
    a2_ref[...] = a2.reshape(_BLK)


_tc_rank = pl.pallas_call(
    _rank_body,
    grid=(_N_BLK,),
    in_specs=[
        pl.BlockSpec((_BLK, _DIM), lambda i: (i, 0)),
        pl.BlockSpec((_N_CODES, _DIM), lambda i: (0, 0)),
    ],
    out_specs=(
        pl.BlockSpec((_BLK,), lambda i: (i,)),
        pl.BlockSpec((_BLK,), lambda i: (i,)),
        pl.BlockSpec((_N_CODES, _PAD_DIM), lambda i: (0, 0)),
    ),
    out_shape=(
        jax.ShapeDtypeStruct((_N_TOKENS,), jnp.int32),
        jax.ShapeDtypeStruct((_N_TOKENS,), jnp.int32),
        jax.ShapeDtypeStruct((_N_CODES, _PAD_DIM), jnp.float32),
    ),
    scratch_shapes=[pltpu.VMEM((1, _N_CODES), jnp.float32)],
)


@functools.cache
def _sc_gather_fn():
    @functools.partial(
        pl.kernel,
        mesh=plsc.VectorSubcoreMesh(core_axis_name="c", subcore_axis_name="s"),
        out_type=(
            jax.ShapeDtypeStruct((_N_TOKENS, _PAD_DIM), jnp.float32),
            jax.ShapeDtypeStruct((_N_TOKENS, _PAD_DIM), jnp.float32),
        ),
        scratch_types=[
            pltpu.VMEM((_ROWS_PER_W,), jnp.int32),
            pltpu.VMEM((_ROWS_PER_W,), jnp.int32),
            pltpu.VMEM((_ROWS_PER_W, _PAD_DIM), jnp.float32),
            pltpu.VMEM((_ROWS_PER_W, _PAD_DIM), jnp.float32),
            pltpu.SemaphoreType.DMA,
            pltpu.SemaphoreType.DMA,
        ],
    )
    def _sc_gather2(cbp_hbm, a1_hbm, a2_hbm, o1_hbm, o2_hbm,
                    i1_v, i2_v, r1_v, r2_v, sem1, sem2):
        wid = lax.axis_index("s") * _NC + lax.axis_index("c")
        base = wid * _ROWS_PER_W
        pltpu.sync_copy(a1_hbm.at[pl.ds(base, _ROWS_PER_W)], i1_v)
        pltpu.sync_copy(a2_hbm.at[pl.ds(base, _ROWS_PER_W)], i2_v)
        cp1 = pltpu.async_copy(cbp_hbm.at[i1_v], r1_v, sem1)
        cp2 = pltpu.async_copy(cbp_hbm.at[i2_v], r2_v, sem2)
        cp1.wait()
        cp2.wait()
        pltpu.sync_copy(r1_v, o1_hbm.at[pl.ds(base, _ROWS_PER_W)])
        pltpu.sync_copy(r2_v, o2_hbm.at[pl.ds(base, _ROWS_PER_W)])

    return _sc_gather2


def _pick_body(x_ref, g1_ref, g2_ref, q_ref):
    x = x_ref[...]
    g1 = g1_ref[...]
    g2 = g2_ref[...]
    c1 = g1[:, :_DIM]
    c2 = g2[:, :_DIM]
    a1 = g1[:, _DIM:_DIM + 1]
    a2 = g2[:, _DIM:_DIM + 1]
    df1 = x - c1
    df2 = x - c2
    d1 = jnp.sqrt(jnp.sum(df1 * df1, axis=1, keepdims=True))
    d2 = jnp.sqrt(jnp.sum(df2 * df2, axis=1, keepdims=True))
    pick2 = (d2 < d1) | ((d2 == d1) & (a2 < a1))
    q_ref[...] = jnp.where(pick2, c2, c1)


_tc_pick = pl.pallas_call(
    _pick_body,
    grid=(_N_BLK,),
    in_specs=[
        pl.BlockSpec((_BLK, _DIM), lambda i: (i, 0)),
        pl.BlockSpec((_BLK, _PAD_DIM), lambda i: (i, 0)),
        pl.BlockSpec((_BLK, _PAD_DIM), lambda i: (i, 0)),
    ],
    out_specs=pl.BlockSpec((_BLK, _DIM), lambda i: (i, 0)),
    out_shape=jax.ShapeDtypeStruct((_N_TOKENS, _DIM), jnp.float32),
)


def kernel(inputs, codebook):
    a1, a2, cbp = _tc_rank(inputs, codebook)
    g1, g2 = _sc_gather_fn()(cbp, a1, a2)
    return _tc_pick(inputs, g1, g2)
